# baseline (speedup 1.0000x reference)
import functools

import jax
import jax.numpy as jnp
from jax import lax
from jax.experimental import pallas as pl
from jax.experimental.pallas import tpu as pltpu
from jax.experimental.pallas import tpu_sc as plsc

MAX_LEN = 4096


@functools.cache
def _make_gather(T: int, D: int, B: int):
    info = plsc.get_sparse_core_info()
    NC, NS = info.num_cores, info.num_subcores
    NW = NC * NS
    assert B % (8 * NW) == 0
    b_per_w = B // NW
    mesh = plsc.VectorSubcoreMesh(core_axis_name="c", subcore_axis_name="s")

    @functools.partial(
        pl.kernel,
        mesh=mesh,
        out_type=jax.ShapeDtypeStruct((B, D), jnp.float32),
        scratch_types=[
            pltpu.VMEM((b_per_w,), jnp.int32),
            pltpu.VMEM((b_per_w, D), jnp.float32),
            pltpu.SemaphoreType.DMA,
            pltpu.SemaphoreType.DMA,
            pltpu.SemaphoreType.DMA,
            pltpu.SemaphoreType.DMA,
        ],
    )
    def k(feature_hbm, idx_hbm, out_hbm, idx_v, rows_v, g0, g1, p0, p1):
        wid = lax.axis_index("s") * NC + lax.axis_index("c")
        base = wid * b_per_w
        h = b_per_w // 2
        pltpu.sync_copy(idx_hbm.at[pl.ds(base, b_per_w)], idx_v)
        ga = pltpu.async_copy(
            feature_hbm.at[idx_v.at[pl.ds(0, h)]], rows_v.at[pl.ds(0, h)], g0)
        gb = pltpu.async_copy(
            feature_hbm.at[idx_v.at[pl.ds(h, h)]], rows_v.at[pl.ds(h, h)], g1)
        ga.wait()
        pa = pltpu.async_copy(
            rows_v.at[pl.ds(0, h)], out_hbm.at[pl.ds(base, h)], p0)
        gb.wait()
        pb = pltpu.async_copy(
            rows_v.at[pl.ds(h, h)], out_hbm.at[pl.ds(base + h, h)], p1)
        pa.wait()
        pb.wait()

    return k


def kernel(feature):
    t, d = feature.shape
    if t <= MAX_LEN:
        return jnp.pad(feature, ((0, MAX_LEN - t), (0, 0)))
    r = jnp.floor(jnp.linspace(0.0, float(t - 1), MAX_LEN)).astype(jnp.int32)
    return _make_gather(t, d, MAX_LEN)(feature, r)


# speedup vs baseline: 1.0140x; 1.0140x over previous
import functools

import jax
import jax.numpy as jnp
from jax import lax
from jax.experimental import pallas as pl
from jax.experimental.pallas import tpu as pltpu
from jax.experimental.pallas import tpu_sc as plsc

MAX_LEN = 4096


@functools.cache
def _make_gather(T: int, D: int, B: int):
    info = plsc.get_sparse_core_info()
    NC, NS = info.num_cores, info.num_subcores
    NW = NC * NS
    assert B % (8 * NW) == 0
    b_per_w = B // NW
    mesh = plsc.VectorSubcoreMesh(core_axis_name="c", subcore_axis_name="s")

    @functools.partial(
        pl.kernel,
        mesh=mesh,
        out_type=jax.ShapeDtypeStruct((B, D), jnp.float32),
        scratch_types=[
            pltpu.VMEM((b_per_w,), jnp.int32),
            pltpu.VMEM((b_per_w, D), jnp.float32),
            pltpu.SemaphoreType.DMA,
        ],
    )
    def k(feature_hbm, idx_hbm, out_hbm, idx_v, rows_v, sem):
        wid = lax.axis_index("s") * NC + lax.axis_index("c")
        base = wid * b_per_w
        pltpu.sync_copy(idx_hbm.at[pl.ds(base, b_per_w)], idx_v)
        pltpu.async_copy(feature_hbm.at[idx_v], rows_v, sem).wait()
        pltpu.sync_copy(rows_v, out_hbm.at[pl.ds(base, b_per_w)])

    return k


def kernel(feature):
    t, d = feature.shape
    if t <= MAX_LEN:
        return jnp.pad(feature, ((0, MAX_LEN - t), (0, 0)))
    r = jnp.floor(jnp.linspace(0.0, float(t - 1), MAX_LEN)).astype(jnp.int32)
    return _make_gather(t, d, MAX_LEN)(feature, r)
